# baseline (speedup 1.0000x reference)
import functools
import jax
import jax.numpy as jnp
import numpy as np
from jax import lax
from jax.experimental import pallas as pl
from jax.experimental.pallas import tpu as pltpu
from jax.experimental.pallas import tpu_sc as plsc

_CHUNK = 16000
_NBUF = 1
_ROWS_PER_W = 2

try:
    _PERM = np.asarray(
        jnp.argsort(jax.random.uniform(jax.random.key(42), (32,)))
    ).tolist()
except Exception:
    _PERM = [22, 18, 6, 26, 21, 27, 10, 20, 24, 4, 31, 14, 0, 3, 5, 17,
             28, 2, 23, 1, 8, 16, 30, 7, 19, 15, 9, 13, 11, 25, 12, 29]
_TBL = _PERM + list(range(32, 64))


def _sc_remix(src_hbm, out_hbm, bufs, rsem, wsem):
    nc = 2
    wid = lax.axis_index("s") * nc + lax.axis_index("c")
    t = src_hbm.shape[3]
    k = t // _CHUNK

    xfers = []
    for r in range(_ROWS_PER_W):
        src_flat = jnp.int32(_TBL[(32 - 1) * _ROWS_PER_W + r])
        for w_cand in reversed(range(32 - 1)):
            src_flat = jnp.where(
                wid == w_cand,
                jnp.int32(_TBL[w_cand * _ROWS_PER_W + r]),
                src_flat,
            )
        b_flat = wid * _ROWS_PER_W + r
        out_s = b_flat // 32
        out_b = b_flat % 32
        src_s = src_flat // 32
        src_b = src_flat % 32
        for j in range(k):
            xfers.append((out_s, out_b, src_s, src_b, j))

    def gather(ti, bb):
        out_s, out_b, src_s, src_b, j = xfers[ti]
        return pltpu.make_async_copy(
            src_hbm.at[pl.ds(src_s, 1), pl.ds(src_b, 1), pl.ds(0, 1),
                       pl.ds(j * _CHUNK, _CHUNK)],
            bufs.at[bb],
            rsem.at[bb],
        )

    def write(ti, bb):
        out_s, out_b, src_s, src_b, j = xfers[ti]
        return pltpu.make_async_copy(
            bufs.at[bb],
            out_hbm.at[pl.ds(out_s, 1), pl.ds(out_b, 1), pl.ds(0, 1),
                       pl.ds(j * _CHUNK, _CHUNK)],
            wsem.at[bb],
        )

    n = len(xfers)
    for ti in range(min(_NBUF, n)):
        gather(ti, ti).start()
    for ti in range(n):
        bb = ti % _NBUF
        if ti >= _NBUF:
            write(ti - _NBUF, bb).wait()
            gather(ti, bb).start()
        gather(ti, bb).wait()
        write(ti, bb).start()
    for ti in range(max(0, n - _NBUF), n):
        write(ti, ti % _NBUF).wait()


def kernel(sources):
    mesh = plsc.VectorSubcoreMesh(core_axis_name="c", subcore_axis_name="s")

    k = functools.partial(
        pl.kernel,
        mesh=mesh,
        out_type=jax.ShapeDtypeStruct(sources.shape, sources.dtype),
        scratch_types=[
            pltpu.VMEM((_NBUF, 1, 1, 1, _CHUNK), sources.dtype),
            pltpu.SemaphoreType.DMA((_NBUF,)),
            pltpu.SemaphoreType.DMA((_NBUF,)),
        ],
    )(_sc_remix)
    return k(sources)


# speedup vs baseline: 1.2329x; 1.2329x over previous
import functools
import jax
import jax.numpy as jnp
import numpy as np
from jax import lax
from jax.experimental import pallas as pl
from jax.experimental.pallas import tpu as pltpu
from jax.experimental.pallas import tpu_sc as plsc

_CHUNK = 16000
_NBUF = 7
_ROWS_PER_W = 2

try:
    _PERM = np.asarray(
        jnp.argsort(jax.random.uniform(jax.random.key(42), (32,)))
    ).tolist()
except Exception:
    _PERM = [22, 18, 6, 26, 21, 27, 10, 20, 24, 4, 31, 14, 0, 3, 5, 17,
             28, 2, 23, 1, 8, 16, 30, 7, 19, 15, 9, 13, 11, 25, 12, 29]
_TBL = _PERM + list(range(32, 64))


def _sc_remix(src_hbm, out_hbm, bufs, rsem, wsem):
    nc = 2
    wid = lax.axis_index("s") * nc + lax.axis_index("c")
    t = src_hbm.shape[3]
    k = t // _CHUNK

    xfers = []
    for r in range(_ROWS_PER_W):
        src_flat = jnp.int32(_TBL[(32 - 1) * _ROWS_PER_W + r])
        for w_cand in reversed(range(32 - 1)):
            src_flat = jnp.where(
                wid == w_cand,
                jnp.int32(_TBL[w_cand * _ROWS_PER_W + r]),
                src_flat,
            )
        b_flat = wid * _ROWS_PER_W + r
        out_s = b_flat // 32
        out_b = b_flat % 32
        src_s = src_flat // 32
        src_b = src_flat % 32
        for j in range(k):
            xfers.append((out_s, out_b, src_s, src_b, j))

    def gather(ti, bb):
        out_s, out_b, src_s, src_b, j = xfers[ti]
        return pltpu.make_async_copy(
            src_hbm.at[pl.ds(src_s, 1), pl.ds(src_b, 1), pl.ds(0, 1),
                       pl.ds(j * _CHUNK, _CHUNK)],
            bufs.at[bb],
            rsem.at[bb],
        )

    def write(ti, bb):
        out_s, out_b, src_s, src_b, j = xfers[ti]
        return pltpu.make_async_copy(
            bufs.at[bb],
            out_hbm.at[pl.ds(out_s, 1), pl.ds(out_b, 1), pl.ds(0, 1),
                       pl.ds(j * _CHUNK, _CHUNK)],
            wsem.at[bb],
        )

    n = len(xfers)
    for ti in range(min(_NBUF, n)):
        gather(ti, ti).start()
    for ti in range(n):
        bb = ti % _NBUF
        gather(ti, bb).wait()
        write(ti, bb).start()
        if ti + _NBUF < n:
            write(ti, bb).wait()
            gather(ti + _NBUF, bb).start()
    for ti in range(max(0, n - _NBUF), n):
        write(ti, ti % _NBUF).wait()


def kernel(sources):
    mesh = plsc.VectorSubcoreMesh(core_axis_name="c", subcore_axis_name="s")

    k = functools.partial(
        pl.kernel,
        mesh=mesh,
        out_type=jax.ShapeDtypeStruct(sources.shape, sources.dtype),
        scratch_types=[
            pltpu.VMEM((_NBUF, 1, 1, 1, _CHUNK), sources.dtype),
            pltpu.SemaphoreType.DMA((_NBUF,)),
            pltpu.SemaphoreType.DMA((_NBUF,)),
        ],
    )(_sc_remix)
    return k(sources)
